# baseline (speedup 1.0000x reference)
import functools

import jax
import jax.numpy as jnp
from jax import lax
from jax.experimental import pallas as pl
from jax.experimental.pallas import tpu as pltpu
from jax.experimental.pallas import tpu_sc as plsc

D = 64
LANES = 16
NUM_WORKERS = 32
BB = 512


def _embed_kernel(idx_hbm, tpair_hbm, out_hbm, idx_v, hi_v, par_v, scale_v,
                  pairs_v, stage_v, sem, *, num_s, batch):
    wid = lax.axis_index("s") * 2 + lax.axis_index("c")
    blocks_per_s = batch // BB
    tasks = num_s * blocks_per_s // NUM_WORKERS
    lane_iota = lax.iota(jnp.int32, LANES)

    @pl.loop(0, tasks)
    def _task(k):
        t = wid * tasks + k
        s = t // blocks_per_s
        b0 = (t % blocks_per_s) * BB
        pltpu.sync_copy(idx_hbm.at[s, pl.ds(b0, BB)], idx_v)

        @pl.loop(0, BB // LANES)
        def _prep(g):
            sl = pl.ds(g * LANES, LANES)
            iv = idx_v[sl]
            hi_v[sl] = iv >> 1
            par_v[sl] = (iv & 1) * D
            scale_v[sl] = jnp.where(iv != 0, jnp.float32(8.0), jnp.float32(0.0))

        pltpu.async_copy(tpair_hbm.at[hi_v], pairs_v, sem).wait()

        @pl.loop(0, BB // LANES)
        def _group(g):
            bl = g * LANES
            sl = pl.ds(bl, LANES)
            sc = scale_v[sl]
            par = par_v[sl]
            b_vec = lane_iota + bl
            for d in range(D):
                v = plsc.load_gather(pairs_v, [b_vec, par + d])
                stage_v[d, sl] = v * sc

        pltpu.sync_copy(stage_v, out_hbm.at[s, :, pl.ds(b0, BB)])


def kernel(input_sequence, table):
    B, S = input_sequence.shape
    V, d = table.shape
    assert d == D and B % BB == 0 and V % 2 == 0
    assert (S * (B // BB)) % NUM_WORKERS == 0
    idx_t = input_sequence.astype(jnp.int32).T
    tpair = table.reshape(V // 2, 2 * D)

    mesh = plsc.VectorSubcoreMesh(core_axis_name="c", subcore_axis_name="s")
    out = pl.kernel(
        functools.partial(_embed_kernel, num_s=S, batch=B),
        out_type=jax.ShapeDtypeStruct((S, D, B), jnp.float32),
        mesh=mesh,
        compiler_params=pltpu.CompilerParams(
            needs_layout_passes=False, use_tc_tiling_on_sc=True
        ),
        scratch_types=[
            pltpu.VMEM((BB,), jnp.int32),
            pltpu.VMEM((BB,), jnp.int32),
            pltpu.VMEM((BB,), jnp.int32),
            pltpu.VMEM((BB,), jnp.float32),
            pltpu.VMEM((BB, 2 * D), jnp.float32),
            pltpu.VMEM((D, BB), jnp.float32),
            pltpu.SemaphoreType.DMA,
        ],
    )(idx_t, tpair)
    return out.transpose(2, 0, 1)


# speedup vs baseline: 1.3562x; 1.3562x over previous
import functools

import jax
import jax.numpy as jnp
from jax import lax
from jax.experimental import pallas as pl
from jax.experimental.pallas import tpu as pltpu
from jax.experimental.pallas import tpu_sc as plsc

D = 64
LANES = 16
NUM_WORKERS = 32
BB = 512


def _embed_kernel(idx_hbm, table_hbm, out_hbm, idx_v, scale_v, rows_v, pad_v,
                  stage_v, sem, *, num_s, batch):
    wid = lax.axis_index("s") * 2 + lax.axis_index("c")
    blocks_per_s = batch // BB
    tasks = num_s * blocks_per_s // NUM_WORKERS
    lane_iota = lax.iota(jnp.int32, LANES)
    col17 = lane_iota * 17

    @pl.loop(0, tasks)
    def _task(k):
        t = wid * tasks + k
        s = t // blocks_per_s
        bblk = t % blocks_per_s
        pltpu.sync_copy(idx_hbm.at[s, pl.ds(bblk * BB, BB)], idx_v)

        @pl.loop(0, BB // LANES)
        def _prep(g):
            sl = pl.ds(g * LANES, LANES)
            scale_v[sl] = jnp.where(idx_v[sl] != 0, jnp.float32(8.0),
                                    jnp.float32(0.0))

        pltpu.async_copy(table_hbm.at[idx_v], rows_v, sem).wait()

        @pl.loop(0, BB // LANES)
        def _g(g):
            bl = g * LANES
            sc = scale_v[pl.ds(bl, LANES)]
            btile = bl // 128
            brem = bl % 128
            for j in range(D // LANES):
                for r in range(LANES):
                    pad_v[pl.ds(r * 17, LANES)] = (
                        rows_v[bl + r, pl.ds(j * LANES, LANES)]
                    )
                for c in range(LANES):
                    d = j * LANES + c
                    v = plsc.load_gather(pad_v, [col17 + c])
                    row = (d // 8) * 32 + btile * 8 + (d % 8)
                    stage_v[row, pl.ds(brem, LANES)] = v * sc

        for dblk in range(D // 8):
            row0 = (s * 8 + dblk) * (batch // 128) + bblk * (BB // 128)
            pltpu.sync_copy(
                stage_v.at[pl.ds(dblk * (BB // 128) * 8, (BB // 128) * 8), :],
                out_hbm.at[pl.ds(row0 * 8, (BB // 128) * 8), :],
            )


def kernel(input_sequence, table):
    B, S = input_sequence.shape
    V, d = table.shape
    assert d == D and B % BB == 0 and BB % 128 == 0
    assert (S * (B // BB)) % NUM_WORKERS == 0
    idx_t = input_sequence.astype(jnp.int32).T

    mesh = plsc.VectorSubcoreMesh(core_axis_name="c", subcore_axis_name="s")
    nrows = S * (D // 8) * (B // 128) * 8
    out = pl.kernel(
        functools.partial(_embed_kernel, num_s=S, batch=B),
        out_type=jax.ShapeDtypeStruct((nrows, 128), jnp.float32),
        mesh=mesh,
        compiler_params=pltpu.CompilerParams(
            needs_layout_passes=False, use_tc_tiling_on_sc=False
        ),
        scratch_types=[
            pltpu.VMEM((BB,), jnp.int32),
            pltpu.VMEM((BB,), jnp.float32),
            pltpu.VMEM((BB, D), jnp.float32),
            pltpu.VMEM((LANES * 17,), jnp.float32),
            pltpu.VMEM(((D // 8) * (BB // 128) * 8, 128), jnp.float32),
            pltpu.SemaphoreType.DMA,
        ],
    )(idx_t, table)
    out5 = out.reshape(S, D // 8, B // 128, 8, 128)
    return out5.transpose(2, 4, 0, 1, 3).reshape(B, S, D)


# speedup vs baseline: 1.4996x; 1.1057x over previous
import functools

import jax
import jax.numpy as jnp
from jax import lax
from jax.experimental import pallas as pl
from jax.experimental.pallas import tpu as pltpu
from jax.experimental.pallas import tpu_sc as plsc

D = 64
LANES = 16
NUM_WORKERS = 32
BB = 256
NBUF = 2


def _embed_kernel(idx_hbm, table_hbm, out_hbm, idx_v, scale_v, rows_v, pad_v,
                  stage_v, sems, semo, *, num_s, batch):
    wid = lax.axis_index("s") * 2 + lax.axis_index("c")
    blocks_per_s = batch // BB
    btiles = BB // 128
    srows = (D // 8) * btiles * 8
    tasks = num_s * blocks_per_s // NUM_WORKERS
    lane_iota = lax.iota(jnp.int32, LANES)
    col17 = lane_iota * 17

    def task_sb(k):
        t = wid * tasks + k
        return t // blocks_per_s, t % blocks_per_s

    def fetch(k, b):
        @pl.when(k < tasks)
        def _():
            s, bblk = task_sb(k)
            pltpu.sync_copy(idx_hbm.at[s, pl.ds(bblk * BB, BB)], idx_v.at[b])
            pltpu.async_copy(table_hbm.at[idx_v.at[b]], rows_v.at[b],
                             sems.at[b])

    def out_copies(k, b):
        s, bblk = task_sb(k)
        for dblk in range(D // 8):
            row0 = ((s * 8 + dblk) * (batch // 128) + bblk * btiles) * 8
            yield pltpu.make_async_copy(
                stage_v.at[b, pl.ds(dblk * btiles * 8, btiles * 8), :],
                out_hbm.at[pl.ds(row0, btiles * 8), :],
                semo.at[b],
            )

    for b in range(NBUF):
        fetch(b, b)

    @pl.loop(0, tasks, step=NBUF)
    def _ring(k0):
        for b in range(NBUF):
            k = k0 + b
            s, bblk = task_sb(k)
            pltpu.make_async_copy(table_hbm.at[idx_v.at[b]], rows_v.at[b],
                                  sems.at[b]).wait()

            @pl.when(k >= NBUF)
            def _():
                for c in out_copies(k - NBUF, b):
                    c.wait()

            @pl.loop(0, BB // LANES)
            def _prep(g):
                sl = pl.ds(g * LANES, LANES)
                scale_v[sl] = jnp.where(idx_v[b, sl] != 0, jnp.float32(8.0),
                                        jnp.float32(0.0))

            @pl.loop(0, BB // LANES)
            def _g(g):
                bl = g * LANES
                sc = scale_v[pl.ds(bl, LANES)]
                btile = bl // 128
                brem = bl % 128
                for j in range(D // LANES):
                    for r in range(LANES):
                        pad_v[pl.ds(r * 17, LANES)] = (
                            rows_v[b, bl + r, pl.ds(j * LANES, LANES)]
                        )
                    for c in range(LANES):
                        d = j * LANES + c
                        v = plsc.load_gather(pad_v, [col17 + c])
                        row = (d // 8) * (btiles * 8) + btile * 8 + (d % 8)
                        stage_v[b, row, pl.ds(brem, LANES)] = v * sc

            for c in out_copies(k, b):
                c.start()
            fetch(k + NBUF, b)

    for b in range(NBUF):
        for c in out_copies(tasks - NBUF + b, b):
            c.wait()


def kernel(input_sequence, table):
    B, S = input_sequence.shape
    V, d = table.shape
    assert d == D and B % BB == 0
    assert (S * (B // BB)) % NUM_WORKERS == 0
    idx_t = input_sequence.astype(jnp.int32).T

    mesh = plsc.VectorSubcoreMesh(core_axis_name="c", subcore_axis_name="s")
    nrows = S * (D // 8) * (B // 128) * 8
    srows = (D // 8) * (BB // 128) * 8
    out = pl.kernel(
        functools.partial(_embed_kernel, num_s=S, batch=B),
        out_type=jax.ShapeDtypeStruct((nrows, 128), jnp.float32),
        mesh=mesh,
        compiler_params=pltpu.CompilerParams(
            needs_layout_passes=False, use_tc_tiling_on_sc=False
        ),
        scratch_types=[
            pltpu.VMEM((NBUF, BB), jnp.int32),
            pltpu.VMEM((BB,), jnp.float32),
            pltpu.VMEM((NBUF, BB, D), jnp.float32),
            pltpu.VMEM((LANES * 17,), jnp.float32),
            pltpu.VMEM((NBUF, srows, 128), jnp.float32),
            pltpu.SemaphoreType.DMA((NBUF,)),
            pltpu.SemaphoreType.DMA((NBUF,)),
        ],
    )(idx_t, table)
    out5 = out.reshape(S, D // 8, B // 128, 8, 128)
    return out5.transpose(2, 4, 0, 1, 3).reshape(B, S, D)
